# baseline (speedup 1.0000x reference)
import functools

import jax
import jax.numpy as jnp
from jax import lax
from jax.experimental import pallas as pl
from jax.experimental.pallas import tpu as pltpu
from jax.experimental.pallas import tpu_sc as plsc

BATCH = 16384
DIM = 64
GRP = 8
BLK = 32


@jax.jit
def _gather(idx, table):
    info = plsc.get_sparse_core_info()
    nc, ns = info.num_cores, info.num_subcores
    nw = nc * ns
    b_per_w = BATCH // nw
    nblk = b_per_w // BLK
    mesh = plsc.VectorSubcoreMesh(core_axis_name="c", subcore_axis_name="s")

    @functools.partial(
        pl.kernel,
        mesh=mesh,
        out_type=jax.ShapeDtypeStruct((BATCH, DIM), jnp.float32),
        scratch_types=[
            pltpu.VMEM((b_per_w,), jnp.int32),
            pltpu.VMEM((b_per_w,), jnp.int32),
            pltpu.VMEM((2, BLK, GRP, DIM), jnp.float32),
            pltpu.VMEM((2, BLK, DIM), jnp.float32),
            pltpu.SemaphoreType.DMA,
            pltpu.SemaphoreType.DMA,
        ],
    )
    def k(idx_hbm, table_hbm, out_hbm, idx_v, rem_v, ring, oblk,
          sem0, sem1):
        wid = lax.axis_index("s") * nc + lax.axis_index("c")
        base = wid * b_per_w
        pltpu.sync_copy(idx_hbm.at[pl.ds(base, b_per_w)], idx_v)

        def split(i, _):
            v = idx_v[pl.ds(i * 16, 16)]
            idx_v[pl.ds(i * 16, 16)] = lax.bitwise_and(v, jnp.int32(~7))
            rem_v[pl.ds(i * 16, 16)] = lax.bitwise_and(v, 7)
            return _

        lax.fori_loop(0, b_per_w // 16, split, 0)

        def fire(blk, half, sem):
            j0 = blk * BLK
            for t in range(BLK // 16):
                gvec = idx_v[pl.ds(j0 + t * 16, 16)]
                for u in range(16):
                    g8 = pl.multiple_of(gvec[u], GRP)
                    pltpu.async_copy(
                        table_hbm.at[pl.ds(g8, GRP)],
                        ring.at[half, t * 16 + u],
                        sem,
                    )

        def wait_blk(sem):
            for _ in range(BLK):
                pltpu.make_async_copy(
                    table_hbm.at[pl.ds(0, GRP)],
                    ring.at[0, 0],
                    sem,
                ).wait()

        def extract(blk, half):
            j0 = blk * BLK
            for t in range(BLK // 16):
                rvec = rem_v[pl.ds(j0 + t * 16, 16)]
                for u in range(16):
                    r8 = rvec[u]
                    for q in range(DIM // 16):
                        oblk[half, t * 16 + u, pl.ds(q * 16, 16)] = ring[
                            half, t * 16 + u, r8, pl.ds(q * 16, 16)
                        ]
            pltpu.sync_copy(oblk.at[half], out_hbm.at[pl.ds(base + j0, BLK)])

        fire(0, 0, sem0)

        def step(s, _):
            fire(2 * s + 1, 1, sem1)
            wait_blk(sem0)
            extract(2 * s, 0)
            fire(2 * s + 2, 0, sem0)
            wait_blk(sem1)
            extract(2 * s + 1, 1)
            return _

        lax.fori_loop(0, nblk // 2 - 1, step, 0)

        fire(nblk - 1, 1, sem1)
        wait_blk(sem0)
        extract(nblk - 2, 0)
        wait_blk(sem1)
        extract(nblk - 1, 1)

    return k(idx, table)


def kernel(idx, table):
    return _gather(idx.astype(jnp.int32), table)


# speedup vs baseline: 1.0680x; 1.0680x over previous
import functools

import jax
import jax.numpy as jnp
from jax import lax
from jax.experimental import pallas as pl
from jax.experimental.pallas import tpu as pltpu
from jax.experimental.pallas import tpu_sc as plsc

BATCH = 16384
DIM = 64
FIRE = 16
NSEM = 8


@jax.jit
def _gather(idx, table):
    info = plsc.get_sparse_core_info()
    nc, ns = info.num_cores, info.num_subcores
    nw = nc * ns
    b_per_w = BATCH // nw
    mesh = plsc.VectorSubcoreMesh(core_axis_name="c", subcore_axis_name="s")

    @functools.partial(
        pl.kernel,
        mesh=mesh,
        out_type=jax.ShapeDtypeStruct((BATCH, DIM), jnp.float32),
        scratch_types=[
            pltpu.VMEM((b_per_w,), jnp.int32),
            pltpu.VMEM((b_per_w, DIM), jnp.float32),
        ]
        + [pltpu.SemaphoreType.DMA] * NSEM,
    )
    def k(idx_hbm, table_hbm, out_hbm, idx_v, rows_v, *sems):
        wid = lax.axis_index("s") * nc + lax.axis_index("c")
        base = wid * b_per_w
        pltpu.sync_copy(idx_hbm.at[pl.ds(base, b_per_w)], idx_v)

        def group(g, _):
            j0 = g * FIRE
            ivec = idx_v[pl.ds(j0, FIRE)]
            for u in range(FIRE):
                r = ivec[u]
                pltpu.async_copy(
                    table_hbm.at[pl.ds(r, 1)],
                    rows_v.at[pl.ds(j0 + u, 1)],
                    sems[u % NSEM],
                )
            return _

        lax.fori_loop(0, b_per_w // FIRE, group, 0)

        def drain(g, _):
            for u in range(NSEM):
                pltpu.make_async_copy(
                    table_hbm.at[pl.ds(0, FIRE // NSEM)],
                    rows_v.at[pl.ds(0, FIRE // NSEM)],
                    sems[u],
                ).wait()
            return _

        lax.fori_loop(0, b_per_w // FIRE, drain, 0)
        pltpu.sync_copy(rows_v, out_hbm.at[pl.ds(base, b_per_w)])

    return k(idx, table)


def kernel(idx, table):
    return _gather(idx.astype(jnp.int32), table)
